# baseline (device time: 17718 ns/iter reference)
import jax
import jax.numpy as jnp
from jax import lax
from jax.experimental import pallas as pl
from jax.experimental.pallas import tpu as pltpu

N_DEV = 4
B, SQ, SKV, HQ, DH = 2, 128, 512, 4, 64
D_MODEL = 512
D_QK = HQ * DH
CH = SKV // N_DEV
NBH = B * HQ


def kernel(x, Wq, K_ext, V_ext, Wo):
    K_t = jnp.transpose(K_ext, (0, 2, 3, 1))
    V_t = jnp.transpose(V_ext, (0, 2, 3, 1))

    def body(x_hbm, wq_hbm, k_hbm, v_hbm, wo_hbm, out_ref,
             x_ref, wq_ref, k_ref, v_ref, wo_ref,
             ctx_send, ctx_recv, st_send, st_recv,
             in_sems, cs_sems, cr_sems, ss_sems, sr_sems):
        my_pos = lax.axis_index("i")
        p_step = [jnp.bitwise_xor(my_pos, 1), 3 - my_pos]

        in_copies = [
            pltpu.make_async_copy(src, dst, in_sems.at[i])
            for i, (src, dst) in enumerate([
                (x_hbm, x_ref), (wq_hbm, wq_ref), (k_hbm, k_ref),
                (v_hbm, v_ref), (wo_hbm, wo_ref)])
        ]
        for c in in_copies:
            c.start()

        barrier_sem = pltpu.get_barrier_semaphore()
        for nbr in p_step:
            pl.semaphore_signal(barrier_sem, inc=1, device_id=(nbr,),
                                device_id_type=pl.DeviceIdType.MESH)
        pl.semaphore_wait(barrier_sem, 2)

        in_copies[0].wait()
        in_copies[1].wait()
        x_flat = x_ref[...].reshape(B * SQ, D_MODEL).astype(jnp.bfloat16)
        q_all = jnp.dot(x_flat, wq_ref[...].astype(jnp.bfloat16),
                        preferred_element_type=jnp.float32)

        in_copies[2].wait()
        in_copies[3].wait()
        kbg = my_pos * 2 + lax.broadcasted_iota(jnp.int32, (CH, SQ), 0) // 64
        qb = lax.broadcasted_iota(jnp.int32, (CH, SQ), 1) // 64
        mask = (qb == kbg) | (kbg == 0) | ((qb + kbg) % 3 == 0)

        ctx_blocks, m_blocks, l_blocks = [], [], []
        for b in range(B):
            for hh in range(HQ):
                q = q_all[b * SQ:(b + 1) * SQ,
                          hh * DH:(hh + 1) * DH].astype(jnp.bfloat16)
                kmat = k_ref[b, hh].astype(jnp.bfloat16)
                vmat = v_ref[b, hh].astype(jnp.bfloat16)
                s = lax.dot_general(
                    kmat, q, (((0,), (1,)), ((), ())),
                    preferred_element_type=jnp.float32) * 0.125
                s = jnp.where(mask, s, -1e9)
                m = jnp.max(s, axis=0, keepdims=True)
                w = jnp.exp(s - m)
                l = jnp.sum(w, axis=0, keepdims=True)
                ctx = lax.dot_general(
                    vmat, w.astype(jnp.bfloat16), (((1,), (0,)), ((), ())),
                    preferred_element_type=jnp.float32)
                ctx_blocks.append(ctx[None])
                m_blocks.append(m[None])
                l_blocks.append(l[None])
        ctx_acc = jnp.concatenate(ctx_blocks, axis=0)
        m_acc = jnp.concatenate(m_blocks, axis=0)
        l_acc = jnp.concatenate(l_blocks, axis=0)

        for s_i in range(2):
            ctx_send[s_i] = ctx_acc.astype(jnp.bfloat16)
            st_send[s_i] = jnp.concatenate([m_acc, l_acc], axis=1)

            ctx_rdma = pltpu.make_async_remote_copy(
                src_ref=ctx_send.at[s_i], dst_ref=ctx_recv.at[s_i],
                send_sem=cs_sems.at[s_i], recv_sem=cr_sems.at[s_i],
                device_id=(p_step[s_i],),
                device_id_type=pl.DeviceIdType.MESH,
            )
            st_rdma = pltpu.make_async_remote_copy(
                src_ref=st_send.at[s_i], dst_ref=st_recv.at[s_i],
                send_sem=ss_sems.at[s_i], recv_sem=sr_sems.at[s_i],
                device_id=(p_step[s_i],),
                device_id_type=pl.DeviceIdType.MESH,
            )
            ctx_rdma.start()
            st_rdma.start()
            ctx_rdma.wait()
            st_rdma.wait()

            ctx_o = ctx_recv[s_i].astype(jnp.float32)
            m_o = st_recv[s_i, :, 0:1, :]
            l_o = st_recv[s_i, :, 1:2, :]

            m_new = jnp.maximum(m_acc, m_o)
            alpha = jnp.exp(m_acc - m_new)
            beta = jnp.exp(m_o - m_new)
            ctx_acc = alpha * ctx_acc + beta * ctx_o
            l_acc = alpha * l_acc + beta * l_o
            m_acc = m_new

        ctx_n = (ctx_acc / l_acc).astype(jnp.bfloat16)
        in_copies[4].wait()
        wo = wo_ref[...].astype(jnp.bfloat16)
        for b in range(B):
            acc = jnp.zeros((SQ, D_MODEL), jnp.float32)
            for hh in range(HQ):
                acc = acc + lax.dot_general(
                    ctx_n[b * HQ + hh], wo[hh * DH:(hh + 1) * DH, :],
                    (((0,), (0,)), ((), ())),
                    preferred_element_type=jnp.float32)
            out_ref[b, :, :] = acc

    return pl.pallas_call(
        body,
        out_shape=jax.ShapeDtypeStruct((B, SQ, D_MODEL), jnp.float32),
        in_specs=[pl.BlockSpec(memory_space=pl.ANY)] * 5,
        out_specs=pl.BlockSpec(memory_space=pltpu.VMEM),
        scratch_shapes=[
            pltpu.VMEM((B, SQ, D_MODEL), jnp.float32),
            pltpu.VMEM((D_MODEL, D_QK), jnp.float32),
            pltpu.VMEM((B, HQ, DH, CH), jnp.float32),
            pltpu.VMEM((B, HQ, DH, CH), jnp.float32),
            pltpu.VMEM((D_QK, D_MODEL), jnp.float32),
            pltpu.VMEM((2, NBH, DH, SQ), jnp.bfloat16),
            pltpu.VMEM((2, NBH, DH, SQ), jnp.bfloat16),
            pltpu.VMEM((2, NBH, 2, SQ), jnp.float32),
            pltpu.VMEM((2, NBH, 2, SQ), jnp.float32),
            pltpu.SemaphoreType.DMA((5,)),
            pltpu.SemaphoreType.DMA((2,)),
            pltpu.SemaphoreType.DMA((2,)),
            pltpu.SemaphoreType.DMA((2,)),
            pltpu.SemaphoreType.DMA((2,)),
        ],
        compiler_params=pltpu.CompilerParams(collective_id=0),
    )(x, Wq, K_t, V_t, Wo)


# device time: 14847 ns/iter; 1.1934x vs baseline; 1.1934x over previous
import jax
import jax.numpy as jnp
from jax import lax
from jax.experimental import pallas as pl
from jax.experimental.pallas import tpu as pltpu

N_DEV = 4
B, SQ, SKV, HQ, DH = 2, 128, 512, 4, 64
D_MODEL = 512
D_QK = HQ * DH
CH = SKV // N_DEV
NBH = B * HQ


def kernel(x, Wq, K_ext, V_ext, Wo):
    def body(x_ref, wq_ref, k_ref, v_ref, wo_ref, out_ref,
             pay_send, pay_recv, ps_sems, pr_sems):
        my_pos = lax.axis_index("i")
        p_step = [jnp.bitwise_xor(my_pos, 1), 3 - my_pos]

        barrier_sem = pltpu.get_barrier_semaphore()
        for nbr in p_step:
            pl.semaphore_signal(barrier_sem, inc=1, device_id=(nbr,),
                                device_id_type=pl.DeviceIdType.MESH)
        pl.semaphore_wait(barrier_sem, 2)

        x_flat = x_ref[...].reshape(B * SQ, D_MODEL).astype(jnp.bfloat16)
        q_all = jnp.dot(x_flat, wq_ref[...].astype(jnp.bfloat16),
                        preferred_element_type=jnp.float32)

        k_loc = k_ref[...].astype(jnp.bfloat16).reshape(B * CH, D_QK)
        v_loc = v_ref[...].astype(jnp.bfloat16).reshape(B * CH, D_QK)

        kbg = my_pos * 2 + lax.broadcasted_iota(jnp.int32, (CH, SQ), 0) // 64
        qb = lax.broadcasted_iota(jnp.int32, (CH, SQ), 1) // 64
        mask = (qb == kbg) | (kbg == 0) | ((qb + kbg) % 3 == 0)

        ctx_blocks, m_blocks, l_blocks = [], [], []
        for b in range(B):
            for hh in range(HQ):
                q = q_all[b * SQ:(b + 1) * SQ,
                          hh * DH:(hh + 1) * DH].astype(jnp.bfloat16)
                kmat = k_loc[b * CH:(b + 1) * CH, hh * DH:(hh + 1) * DH]
                vmat = v_loc[b * CH:(b + 1) * CH, hh * DH:(hh + 1) * DH]
                s = lax.dot_general(
                    kmat, q, (((1,), (1,)), ((), ())),
                    preferred_element_type=jnp.float32) * 0.125
                s = jnp.where(mask, s, -1e9)
                m = jnp.max(s, axis=0, keepdims=True)
                w = jnp.exp(s - m)
                l = jnp.sum(w, axis=0, keepdims=True)
                ctx = lax.dot_general(
                    vmat, w.astype(jnp.bfloat16), (((0,), (0,)), ((), ())),
                    preferred_element_type=jnp.float32)
                ctx_blocks.append(ctx[None])
                m_blocks.append(m[None])
                l_blocks.append(l[None])
        ctx_acc = jnp.concatenate(ctx_blocks, axis=0)
        m_acc = jnp.concatenate(m_blocks, axis=0)
        l_acc = jnp.concatenate(l_blocks, axis=0)

        rdmas = []
        for s_i in range(2):
            pay_send[s_i, :, :DH, :] = ctx_acc.astype(jnp.bfloat16)
            pay_send[s_i, :, DH:DH + 1, :] = m_acc.astype(jnp.bfloat16)
            pay_send[s_i, :, DH + 1:DH + 2, :] = l_acc.astype(jnp.bfloat16)

            rdma = pltpu.make_async_remote_copy(
                src_ref=pay_send.at[s_i], dst_ref=pay_recv.at[s_i],
                send_sem=ps_sems.at[s_i], recv_sem=pr_sems.at[s_i],
                device_id=(p_step[s_i],),
                device_id_type=pl.DeviceIdType.MESH,
            )
            rdma.start()
            rdma.wait_recv()
            rdmas.append(rdma)

            ctx_o = pay_recv[s_i, :, :DH, :].astype(jnp.float32)
            m_o = pay_recv[s_i, :, DH:DH + 1, :].astype(jnp.float32)
            l_o = pay_recv[s_i, :, DH + 1:DH + 2, :].astype(jnp.float32)

            m_new = jnp.maximum(m_acc, m_o)
            alpha = jnp.exp(m_acc - m_new)
            beta = jnp.exp(m_o - m_new)
            ctx_acc = alpha * ctx_acc + beta * ctx_o
            l_acc = alpha * l_acc + beta * l_o
            m_acc = m_new

        ctx_n = (ctx_acc / l_acc).astype(jnp.bfloat16)
        wo = wo_ref[...].astype(jnp.bfloat16)
        for b in range(B):
            acc = jnp.zeros((SQ, D_MODEL), jnp.float32)
            for hh in range(HQ):
                acc = acc + lax.dot_general(
                    ctx_n[b * HQ + hh], wo[hh * DH:(hh + 1) * DH, :],
                    (((0,), (0,)), ((), ())),
                    preferred_element_type=jnp.float32)
            out_ref[b, :, :] = acc.astype(jnp.bfloat16)

        for rdma in rdmas:
            rdma.wait_send()

    return pl.pallas_call(
        body,
        out_shape=jax.ShapeDtypeStruct((B, SQ, D_MODEL), jnp.bfloat16),
        in_specs=[pl.BlockSpec(memory_space=pltpu.VMEM)] * 5,
        out_specs=pl.BlockSpec(memory_space=pltpu.VMEM),
        scratch_shapes=[
            pltpu.VMEM((2, NBH, DH + 2, SQ), jnp.bfloat16),
            pltpu.VMEM((2, NBH, DH + 2, SQ), jnp.bfloat16),
            pltpu.SemaphoreType.DMA((2,)),
            pltpu.SemaphoreType.DMA((2,)),
        ],
        compiler_params=pltpu.CompilerParams(collective_id=0),
    )(x, Wq, K_ext, V_ext, Wo)
